# baseline (speedup 1.0000x reference)
import math

import jax
import jax.numpy as jnp
from jax.experimental import pallas as pl
from jax.experimental.pallas import tpu as pltpu

BLK = 1024


def _fused_kernel(pos_idx_ref, pos_table_ref, mem_keys_ref, age_ref, conf_ref,
                  xa_ref, xp_ref, wk_ref, mv_ref, wg_ref, bg_ref, wo_ref,
                  bo_ref, gamma_ref, beta_ref, out_ref, kwp_ref, bias_ref,
                  cat_ref, attn_ref):
    h = xa_ref.shape[1]
    s, kd = kwp_ref.shape
    dn = (((1,), (1,)), ((), ()))
    dnr = (((1,), (0,)), ((), ()))
    i = pl.program_id(0)

    @pl.when(i == 0)
    def _prep():
        age = age_ref[...]
        recency = jnp.exp(age * (-1.0 / 200.0))
        freq = jnp.maximum(age, 1.0)
        fmax = jnp.max(freq)
        freq_norm = jnp.log(freq + 1.0) / (jnp.log(fmax + 2.0) + 1e-8)
        bias_ref[...] = (0.2 * recency + 0.15 * freq_norm
                         + 0.1 * conf_ref[...] + 0.08)

        raw = pos_idx_ref[...]
        idx = (raw & (s - 1)) if s & (s - 1) == 0 else raw
        row_j = jax.lax.broadcasted_iota(jnp.int32, (s, s), 0)
        onehot_t = (row_j == idx).astype(jnp.float32)
        pos_emb = jax.lax.dot_general(onehot_t, pos_table_ref[...],
                                      (((0,), (0,)), ((), ())))
        kwp = mem_keys_ref[...] + 0.1 * pos_emb
        norm = jnp.sqrt(jnp.sum(kwp * kwp, axis=-1, keepdims=True))
        kwp_ref[...] = kwp / jnp.maximum(norm, 1e-12)

        mv = mv_ref[...]
        cat_ref[:, :h] = jax.lax.dot_general(mv, wg_ref[...], dn)
        cat_ref[:, h:] = jax.lax.dot_general(mv, wo_ref[...], dn)

    xa = xa_ref[...]
    q = jax.lax.dot_general(xa, wk_ref[...], dn)
    qn = q / jnp.maximum(jnp.sqrt(jnp.sum(q * q, axis=-1, keepdims=True)), 1e-12)
    sim = jax.lax.dot_general(qn, kwp_ref[...], dn) * (1.0 / math.sqrt(kd))
    sal = jnp.clip(0.45 * sim + bias_ref[...], 0.0, 1.0)
    e = jnp.exp(sal)
    attn_ref[i & 1] = e / jnp.sum(e, axis=-1, keepdims=True)

    x = xp_ref[...]
    attn = attn_ref[(i + 1) & 1]
    r = jax.lax.dot_general(attn, mv_ref[...], dnr)
    go = jax.lax.dot_general(attn, cat_ref[...], dnr)
    g = jax.nn.sigmoid(jax.lax.dot_general(x, wg_ref[...], dn)
                       + go[:, :h] + bg_ref[...])
    u = (jax.lax.dot_general(x, wo_ref[...], dn)
         + go[:, h:] + bo_ref[...])
    o = 0.5 * u * (1.0 + jax.lax.erf(u * (1.0 / math.sqrt(2.0))))
    hh = o + x + g * (r - x)
    mu = jnp.mean(hh, axis=-1, keepdims=True)
    hc = hh - mu
    var = jnp.mean(hc * hc, axis=-1, keepdims=True)
    out_ref[...] = hc * jax.lax.rsqrt(var + 1e-5) * gamma_ref[...] + beta_ref[...]


def kernel(x, Wk, Wv, pos_table, Wg, bg, Wo, bo, gamma, beta, mem_keys,
           mem_vals, mem_age, mem_conf, slot_order):
    del Wv
    b, h = x.shape
    s, kd = mem_keys.shape
    nblk = b // BLK

    pos_idx = slot_order.astype(jnp.int32).reshape(1, s)
    if s & (s - 1) != 0:
        pos_idx = pos_idx % s
    const = lambda i: (0, 0)
    att_ix = lambda i: (jnp.minimum(i, nblk - 1), 0)
    proj_ix = lambda i: (jnp.maximum(i - 1, 0), 0)
    half_ix = lambda i: (0, jnp.where(i == 0, 1, 0))
    out = pl.pallas_call(
        _fused_kernel,
        grid=(nblk + 1,),
        in_specs=[
            pl.BlockSpec((1, s), const),
            pl.BlockSpec((s, kd), const),
            pl.BlockSpec((s, kd), const),
            pl.BlockSpec((1, s), const),
            pl.BlockSpec((1, s), const),
            pl.BlockSpec((BLK, h), att_ix),
            pl.BlockSpec((BLK, h), proj_ix),
            pl.BlockSpec((kd, h), const),
            pl.BlockSpec((s, h), const),
            pl.BlockSpec((h, h), half_ix),
            pl.BlockSpec((1, h), const),
            pl.BlockSpec((h, h), half_ix),
            pl.BlockSpec((1, h), const),
            pl.BlockSpec((1, h), const),
            pl.BlockSpec((1, h), const),
        ],
        out_specs=pl.BlockSpec((BLK, h), proj_ix),
        out_shape=jax.ShapeDtypeStruct((b, h), jnp.float32),
        compiler_params=pltpu.CompilerParams(
            vmem_limit_bytes=67043328),
        scratch_shapes=[pltpu.VMEM((s, kd), jnp.float32),
                        pltpu.VMEM((1, s), jnp.float32),
                        pltpu.VMEM((s, 2 * h), jnp.float32),
                        pltpu.VMEM((2, BLK, s), jnp.float32)],
    )(pos_idx, pos_table, mem_keys, mem_age.reshape(1, s),
      mem_conf.reshape(1, s), x, x, Wk, mem_vals, Wg, bg.reshape(1, h), Wo,
      bo.reshape(1, h), gamma.reshape(1, h), beta.reshape(1, h))
    return out


# speedup vs baseline: 1.1418x; 1.1418x over previous
import math

import jax
import jax.numpy as jnp
from jax.experimental import pallas as pl
from jax.experimental.pallas import tpu as pltpu

BLK = 1024
BF = jnp.bfloat16
F32 = jnp.float32


def _fused_kernel(pos_idx_ref, pos_table_ref, mem_keys_ref, age_ref, conf_ref,
                  x_ref, wk_ref, mv_ref, wg_ref, bg_ref, wo_ref, bo_ref,
                  gamma_ref, beta_ref, out_ref, kwp_ref, bias_ref, cat_ref,
                  w1_ref, wkb_ref):
    h = x_ref.shape[1]
    s, kd = kwp_ref.shape
    dn = (((1,), (1,)), ((), ()))
    dnr = (((1,), (0,)), ((), ()))

    @pl.when(pl.program_id(0) == 0)
    def _prep():
        age = age_ref[...]
        recency = jnp.exp(age * (-1.0 / 200.0))
        freq = jnp.maximum(age, 1.0)
        fmax = jnp.max(freq)
        freq_norm = jnp.log(freq + 1.0) / (jnp.log(fmax + 2.0) + 1e-8)
        bias_ref[...] = (0.2 * recency + 0.15 * freq_norm
                         + 0.1 * conf_ref[...] + 0.08)

        raw = pos_idx_ref[...]
        idx = (raw & (s - 1)) if s & (s - 1) == 0 else raw
        row_j = jax.lax.broadcasted_iota(jnp.int32, (s, s), 0)
        onehot_t = (row_j == idx).astype(F32)
        pos_emb = jax.lax.dot_general(onehot_t, pos_table_ref[...],
                                      (((0,), (0,)), ((), ())))
        kwp = mem_keys_ref[...] + 0.1 * pos_emb
        norm = jnp.sqrt(jnp.sum(kwp * kwp, axis=-1, keepdims=True))
        kwp_ref[...] = kwp / jnp.maximum(norm, 1e-12)

        mv = mv_ref[...]
        wg = wg_ref[...]
        wo = wo_ref[...]
        cat_ref[:, :h] = mv.astype(BF)
        cat_ref[:, h:2 * h] = jax.lax.dot_general(mv, wg[:, h:], dn).astype(BF)
        cat_ref[:, 2 * h:] = jax.lax.dot_general(mv, wo[:, h:], dn).astype(BF)
        w1_ref[:h] = wg[:, :h].astype(BF)
        w1_ref[h:] = wo[:, :h].astype(BF)
        wkb_ref[...] = wk_ref[...].astype(BF)

    x = x_ref[...]
    xb = x.astype(BF)
    q = jax.lax.dot_general(xb, wkb_ref[...], dn,
                            preferred_element_type=F32)
    qn = q / jnp.maximum(jnp.sqrt(jnp.sum(q * q, axis=-1, keepdims=True)), 1e-12)
    sim = jax.lax.dot_general(qn, kwp_ref[...], dn) * (1.0 / math.sqrt(kd))
    sal = jnp.clip(0.45 * sim + bias_ref[...], 0.0, 1.0)
    e = jnp.exp(sal)
    attn = (e / jnp.sum(e, axis=-1, keepdims=True)).astype(BF)
    rgo = jax.lax.dot_general(attn, cat_ref[...], dnr,
                              preferred_element_type=F32)
    r = rgo[:, :h]
    gu = jax.lax.dot_general(xb, w1_ref[...], dn,
                             preferred_element_type=F32)
    g = jax.nn.sigmoid(gu[:, :h] + rgo[:, h:2 * h] + bg_ref[...])
    u = gu[:, h:] + rgo[:, 2 * h:] + bo_ref[...]
    o = 0.5 * u * (1.0 + jax.lax.erf(u * (1.0 / math.sqrt(2.0))))
    hh = o + x + g * (r - x)
    mu = jnp.mean(hh, axis=-1, keepdims=True)
    hc = hh - mu
    var = jnp.mean(hc * hc, axis=-1, keepdims=True)
    out_ref[...] = hc * jax.lax.rsqrt(var + 1e-5) * gamma_ref[...] + beta_ref[...]


def kernel(x, Wk, Wv, pos_table, Wg, bg, Wo, bo, gamma, beta, mem_keys,
           mem_vals, mem_age, mem_conf, slot_order):
    del Wv
    b, h = x.shape
    s, kd = mem_keys.shape

    pos_idx = slot_order.astype(jnp.int32).reshape(1, s)
    if s & (s - 1) != 0:
        pos_idx = pos_idx % s
    const = lambda i: (0, 0)
    out = pl.pallas_call(
        _fused_kernel,
        grid=(b // BLK,),
        in_specs=[
            pl.BlockSpec((1, s), const),
            pl.BlockSpec((s, kd), const),
            pl.BlockSpec((s, kd), const),
            pl.BlockSpec((1, s), const),
            pl.BlockSpec((1, s), const),
            pl.BlockSpec((BLK, h), lambda i: (i, 0)),
            pl.BlockSpec((kd, h), const),
            pl.BlockSpec((s, h), const),
            pl.BlockSpec((h, 2 * h), const),
            pl.BlockSpec((1, h), const),
            pl.BlockSpec((h, 2 * h), const),
            pl.BlockSpec((1, h), const),
            pl.BlockSpec((1, h), const),
            pl.BlockSpec((1, h), const),
        ],
        out_specs=pl.BlockSpec((BLK, h), lambda i: (i, 0)),
        out_shape=jax.ShapeDtypeStruct((b, h), jnp.float32),
        compiler_params=pltpu.CompilerParams(
            vmem_limit_bytes=67043328),
        scratch_shapes=[pltpu.VMEM((s, kd), F32),
                        pltpu.VMEM((1, s), F32),
                        pltpu.VMEM((s, 3 * h), BF),
                        pltpu.VMEM((2 * h, h), BF),
                        pltpu.VMEM((kd, h), BF)],
    )(pos_idx, pos_table, mem_keys, mem_age.reshape(1, s),
      mem_conf.reshape(1, s), x, Wk, mem_vals, Wg, bg.reshape(1, h), Wo,
      bo.reshape(1, h), gamma.reshape(1, h), beta.reshape(1, h))
    return out
